# baseline (speedup 1.0000x reference)
import jax
import jax.numpy as jnp
from jax import lax
from jax.experimental import pallas as pl
from jax.experimental.pallas import tpu as pltpu
from jax.experimental.pallas import tpu_sc as plsc


def plsc_bitcast_tc(x, dtype):
    return jax.lax.bitcast_convert_type(x, dtype)

E = 8
D_MODEL = 768
SEQ = 2048

TOK_TILE = 512
N_TOK_TILES = SEQ // TOK_TILE
ROW_TILE = 256
CAP = 2 * SEQ + E * ROW_TILE
N_ROW_TILES = CAP // ROW_TILE
MM_TILES = 8
NW = 32
TPW = SEQ // NW
NVEC = TPW // 16
DW = D_MODEL // 32


def _route_body(x_ref, wg_ref, xbf_ref, e1_ref, e2_ref, w1_ref, w2_ref,
                r1_ref, r2_ref, cnt_ref, carry_ref):
    i = pl.program_id(0)
    xt = x_ref[...]
    T = xt.shape[0]
    H = D_MODEL // 2
    xbf_ref[...] = pltpu.pack_elementwise(
        [xt[:, :H], xt[:, H:]], packed_dtype=jnp.bfloat16)

    logits = lax.dot_general(wg_ref[...], xt, (((1,), (1,)), ((), ())),
                             preferred_element_type=jnp.float32)

    iota = lax.broadcasted_iota(jnp.int32, (E, T), 0)
    m1 = jnp.max(logits, axis=0, keepdims=True)
    a1 = jnp.min(jnp.where(logits >= m1, iota, E), axis=0, keepdims=True)
    masked = jnp.where(iota == a1, -jnp.inf, logits)
    m2 = jnp.max(masked, axis=0, keepdims=True)
    a2 = jnp.min(jnp.where(masked >= m2, iota, E), axis=0, keepdims=True)

    w1 = 1.0 / (1.0 + jnp.exp(m2 - m1))
    w2 = 1.0 - w1

    iota16 = lax.broadcasted_iota(jnp.int32, (2 * E, T), 0)
    oh1 = (iota16 == a1).astype(jnp.float32)
    oh2 = (iota16 == a2).astype(jnp.float32)
    oh = oh1 + oh2

    tri = (lax.broadcasted_iota(jnp.int32, (T, T), 0) <
           lax.broadcasted_iota(jnp.int32, (T, T), 1)).astype(jnp.float32)
    csum = lax.dot_general(oh, tri, (((1,), (0,)), ((), ())),
                           preferred_element_type=jnp.float32)

    @pl.when(i == 0)
    def _():
        carry_ref[...] = jnp.zeros_like(carry_ref)

    carry = carry_ref[...]
    r1 = jnp.sum(oh1 * (carry + csum), axis=0, keepdims=True)
    r2 = jnp.sum(oh2 * (carry + csum), axis=0, keepdims=True)
    carry_ref[...] = carry + jnp.sum(oh, axis=1, keepdims=True)

    e1_ref[...] = a1.reshape(1, 1, T)
    e2_ref[...] = a2.reshape(1, 1, T)
    w1_ref[...] = w1.reshape(1, 1, T)
    w2_ref[...] = w2.reshape(1, 1, T)
    r1_ref[...] = r1.astype(jnp.int32).reshape(1, 1, T)
    r2_ref[...] = r2.astype(jnp.int32).reshape(1, 1, T)

    ones = jnp.ones((1, T), jnp.float32)
    ctile = lax.dot_general(ones, oh, (((1,), (1,)), ((), ())),
                            preferred_element_type=jnp.float32)

    @pl.when(i == 0)
    def _():
        cnt_ref[...] = jnp.zeros_like(cnt_ref)

    cnt_ref[...] = cnt_ref[...] + ctile.astype(jnp.int32).reshape(1, 1, 2 * E)


def _route(x2d, Wg):
    T = TOK_TILE
    specs3 = pl.BlockSpec((1, 1, T), lambda i: (i, 0, 0))
    shape3i = jax.ShapeDtypeStruct((N_TOK_TILES, 1, T), jnp.int32)
    shape3f = jax.ShapeDtypeStruct((N_TOK_TILES, 1, T), jnp.float32)
    return pl.pallas_call(
        _route_body,
        grid=(N_TOK_TILES,),
        in_specs=[
            pl.BlockSpec((T, D_MODEL), lambda i: (i, 0)),
            pl.BlockSpec((E, D_MODEL), lambda i: (0, 0)),
        ],
        out_specs=[pl.BlockSpec((T, D_MODEL // 2), lambda i: (i, 0)),
                   specs3, specs3, specs3, specs3, specs3, specs3,
                   pl.BlockSpec((1, 1, 2 * E), lambda i: (0, 0, 0))],
        out_shape=[jax.ShapeDtypeStruct((SEQ, D_MODEL // 2), jnp.int32),
                   shape3i, shape3i, shape3f, shape3f, shape3i, shape3i,
                   jax.ShapeDtypeStruct((1, 1, 2 * E), jnp.int32)],
        scratch_shapes=[pltpu.VMEM((2 * E, 1), jnp.float32)],
        compiler_params=pltpu.CompilerParams(
            dimension_semantics=("arbitrary",)),
    )(x2d, Wg)


def _dispatch_body(x_hbm, e1_hbm, e2_hbm, r1_hbm, r2_hbm, cnt_hbm,
                   xs_hbm, pos1_hbm, pos2_hbm, eid_hbm,
                   ev1, ev2, rv1, rv2, cnt_v, offp_v, p1_v, p2_v,
                   delta_v, eid_v, xrows, sem, xsem):
    wid = lax.axis_index("s") * 2 + lax.axis_index("c")
    base = wid * TPW

    xdma = pltpu.async_copy(x_hbm.at[pl.ds(base, TPW)], xrows, xsem)

    c1 = pltpu.async_copy(e1_hbm.at[pl.ds(base, TPW)], ev1, sem)
    c2 = pltpu.async_copy(e2_hbm.at[pl.ds(base, TPW)], ev2, sem)
    c3 = pltpu.async_copy(r1_hbm.at[pl.ds(base, TPW)], rv1, sem)
    c4 = pltpu.async_copy(r2_hbm.at[pl.ds(base, TPW)], rv2, sem)
    c5 = pltpu.async_copy(cnt_hbm, cnt_v, sem)
    c1.wait(); c2.wait(); c3.wait(); c4.wait(); c5.wait()

    c = cnt_v[...]
    pc = ((c + (ROW_TILE - 1)) >> 8) << 8
    incl = jnp.cumsum(pc)
    offp = incl - pc
    offp_v[...] = offp

    for v in range(NVEC):
        sl = pl.ds(16 * v, 16)
        p1_v[sl] = plsc.load_gather(offp_v, [ev1[sl]]) + rv1[sl]
        p2_v[sl] = plsc.load_gather(offp_v, [ev2[sl]]) + rv2[sl]

    pltpu.sync_copy(p1_v, pos1_hbm.at[pl.ds(base, TPW)])
    pltpu.sync_copy(p2_v, pos2_hbm.at[pl.ds(base, TPW)])

    @pl.when(wid == 0)
    def _():
        zeros = jnp.zeros((16,), jnp.int32)
        delta_v[pl.ds(0, 16)] = zeros
        delta_v[pl.ds(16, 16)] = zeros
        ts = offp >> 8
        lane = lax.broadcasted_iota(jnp.int32, (16,), 0)
        mask = (lane >= 1) & (lane < E)
        plsc.addupdate_scatter(delta_v, [ts], jnp.ones((16,), jnp.int32),
                               mask=mask)
        d0 = delta_v[pl.ds(0, 16)]
        c0 = jnp.cumsum(d0)
        s0 = jnp.sum(d0, axis=0)
        c1 = jnp.cumsum(delta_v[pl.ds(16, 16)]) + s0
        eid_v[pl.ds(0, 16)] = c0
        eid_v[pl.ds(16, 16)] = c1
        pltpu.sync_copy(eid_v, eid_hbm)

    xdma.wait()
    s1 = pltpu.async_copy(xrows, xs_hbm.at[p1_v], sem)
    s2 = pltpu.async_copy(xrows, xs_hbm.at[p2_v], sem)
    s1.wait()
    s2.wait()


def _dispatch(xbf, e1, e2, r1, r2, cnt):
    mesh = plsc.VectorSubcoreMesh(core_axis_name="c", subcore_axis_name="s",
                                  num_cores=2, num_subcores=16)
    f = pl.kernel(
        _dispatch_body,
        out_type=[
            jax.ShapeDtypeStruct((CAP, D_MODEL // 2), jnp.int32),
            jax.ShapeDtypeStruct((SEQ,), jnp.int32),
            jax.ShapeDtypeStruct((SEQ,), jnp.int32),
            jax.ShapeDtypeStruct((32,), jnp.int32),
        ],
        mesh=mesh,
        scratch_types=[
            pltpu.VMEM((TPW,), jnp.int32),
            pltpu.VMEM((TPW,), jnp.int32),
            pltpu.VMEM((TPW,), jnp.int32),
            pltpu.VMEM((TPW,), jnp.int32),
            pltpu.VMEM((16,), jnp.int32),
            pltpu.VMEM((16,), jnp.int32),
            pltpu.VMEM((TPW,), jnp.int32),
            pltpu.VMEM((TPW,), jnp.int32),
            pltpu.VMEM((32,), jnp.int32),
            pltpu.VMEM((32,), jnp.int32),
            pltpu.VMEM((TPW, D_MODEL // 2), jnp.int32),
            pltpu.SemaphoreType.DMA,
            pltpu.SemaphoreType.DMA,
        ],
        compiler_params=pltpu.CompilerParams(needs_layout_passes=False),
    )
    return f(xbf, e1, e2, r1, r2, cnt)


def _expert_mm_body(eid_ref, xs_ref, we_ref, be_ref, ys_ref):
    i = pl.program_id(0)
    H = D_MODEL // 2
    w = xs_ref[...]
    xlo = pltpu.unpack_elementwise(
        w, index=0, packed_dtype=jnp.bfloat16, unpacked_dtype=jnp.float32)
    xhi = pltpu.unpack_elementwise(
        w, index=1, packed_dtype=jnp.bfloat16, unpacked_dtype=jnp.float32)
    xf = jnp.concatenate([xlo, xhi], axis=1)
    for j in range(MM_TILES):
        e = eid_ref[MM_TILES * i + j]
        sl = pl.ds(ROW_TILE * j, ROW_TILE)
        y = lax.dot_general(
            xf[ROW_TILE * j:ROW_TILE * (j + 1)], we_ref[e],
            (((1,), (1,)), ((), ())),
            preferred_element_type=jnp.float32) + be_ref[pl.ds(e, 1), :]
        ys_ref[sl] = pltpu.pack_elementwise(
            [y[:, :H], y[:, H:]], packed_dtype=jnp.bfloat16)


def _expert_mm(eid, xs, We, be):
    grid_spec = pltpu.PrefetchScalarGridSpec(
        num_scalar_prefetch=1,
        grid=(N_ROW_TILES // MM_TILES,),
        in_specs=[
            pl.BlockSpec((MM_TILES * ROW_TILE, D_MODEL // 2),
                         lambda i, eid: (i, 0)),
            pl.BlockSpec((E, D_MODEL, D_MODEL), lambda i, eid: (0, 0, 0)),
            pl.BlockSpec((E, D_MODEL), lambda i, eid: (0, 0)),
        ],
        out_specs=pl.BlockSpec((MM_TILES * ROW_TILE, D_MODEL // 2),
                               lambda i, eid: (i, 0)),
    )
    return pl.pallas_call(
        _expert_mm_body,
        grid_spec=grid_spec,
        out_shape=jax.ShapeDtypeStruct((CAP, D_MODEL // 2), jnp.int32),
        compiler_params=pltpu.CompilerParams(
            dimension_semantics=("arbitrary",)),
    )(eid, xs, We, be)


def _combine_body(ys_hbm, pos1_hbm, pos2_hbm, w1_hbm, w2_hbm, out_hbm,
                  p1_v, p2_v, w1_v, w2_v, buf1, buf2, obuf, sem, sem2):
    wid = lax.axis_index("s") * 2 + lax.axis_index("c")
    base = wid * TPW

    c1 = pltpu.async_copy(pos1_hbm.at[pl.ds(base, TPW)], p1_v, sem)
    c2 = pltpu.async_copy(pos2_hbm.at[pl.ds(base, TPW)], p2_v, sem)
    c3 = pltpu.async_copy(w1_hbm.at[pl.ds(base, TPW)], w1_v, sem)
    c4 = pltpu.async_copy(w2_hbm.at[pl.ds(base, TPW)], w2_v, sem)
    c1.wait(); c2.wait(); c3.wait(); c4.wait()

    CH = TPW // 2
    chsl = [pl.ds(0, CH), pl.ds(CH, CH)]
    sems = [sem, sem2]
    descs = []
    for c in range(2):
        descs.append(pltpu.async_copy(
            ys_hbm.at[p1_v.at[chsl[c]]], buf1.at[chsl[c]], sems[c]))
        descs.append(pltpu.async_copy(
            ys_hbm.at[p2_v.at[chsl[c]]], buf2.at[chsl[c]], sems[c]))

    hmask = jnp.int32(-65536)
    H = D_MODEL // 2

    for c in range(2):
        descs[2 * c].wait()
        descs[2 * c + 1].wait()

        @plsc.parallel_loop(CH * c, CH * (c + 1), 1, unroll=4)
        def tok_body(t):
            t16 = jnp.full((16,), t, jnp.int32)
            wt1 = plsc.load_gather(w1_v, [t16])
            wt2 = plsc.load_gather(w2_v, [t16])
            for g in range(DW):
                sl = pl.ds(16 * g, 16)
                v1 = buf1[t, sl]
                v2 = buf2[t, sl]
                lo1 = plsc.bitcast(v1 << 16, jnp.float32)
                hi1 = plsc.bitcast(v1 & hmask, jnp.float32)
                lo2 = plsc.bitcast(v2 << 16, jnp.float32)
                hi2 = plsc.bitcast(v2 & hmask, jnp.float32)
                obuf[t, pl.ds(16 * g, 16)] = wt1 * lo1 + wt2 * lo2
                obuf[t, pl.ds(H + 16 * g, 16)] = wt1 * hi1 + wt2 * hi2
    pltpu.sync_copy(obuf, out_hbm.at[pl.ds(base, TPW)])


def _combine(ys_i32, pos1, pos2, w1, w2):
    mesh = plsc.VectorSubcoreMesh(core_axis_name="c", subcore_axis_name="s",
                                  num_cores=2, num_subcores=16)
    f = pl.kernel(
        _combine_body,
        out_type=jax.ShapeDtypeStruct((SEQ, D_MODEL), jnp.float32),
        mesh=mesh,
        scratch_types=[
            pltpu.VMEM((TPW,), jnp.int32),
            pltpu.VMEM((TPW,), jnp.int32),
            pltpu.VMEM((TPW,), jnp.float32),
            pltpu.VMEM((TPW,), jnp.float32),
            pltpu.VMEM((TPW, D_MODEL // 2), jnp.int32),
            pltpu.VMEM((TPW, D_MODEL // 2), jnp.int32),
            pltpu.VMEM((TPW, D_MODEL), jnp.float32),
            pltpu.SemaphoreType.DMA,
            pltpu.SemaphoreType.DMA,
        ],
        compiler_params=pltpu.CompilerParams(needs_layout_passes=False),
    )
    return f(ys_i32, pos1, pos2, w1, w2)


@jax.jit
def _moe(x2d, Wg, We, be):
    xpk, e1, e2, w1, w2, r1, r2, cnt = _route(x2d, Wg)
    flat = lambda a: a.reshape(-1)
    xs_pk, pos1, pos2, eid = _dispatch(
        xpk, flat(e1), flat(e2), flat(r1), flat(r2), flat(cnt))
    ys_pk = _expert_mm(eid[:N_ROW_TILES], xs_pk, We, be)
    return _combine(ys_pk, pos1, pos2, flat(w1), flat(w2))


def kernel(x, Wg, We, be):
    B, S, D = x.shape
    out = _moe(x.reshape(S, D), Wg, We, be)
    return out.reshape(B, S, D)


# speedup vs baseline: 1.0067x; 1.0067x over previous
import jax
import jax.numpy as jnp
from jax import lax
from jax.experimental import pallas as pl
from jax.experimental.pallas import tpu as pltpu
from jax.experimental.pallas import tpu_sc as plsc


def plsc_bitcast_tc(x, dtype):
    return jax.lax.bitcast_convert_type(x, dtype)

E = 8
D_MODEL = 768
SEQ = 2048

TOK_TILE = 512
N_TOK_TILES = SEQ // TOK_TILE
ROW_TILE = 256
CAP = 2 * SEQ + E * ROW_TILE
N_ROW_TILES = CAP // ROW_TILE
MM_TILES = 8
NW = 32
TPW = SEQ // NW
NVEC = TPW // 16
DW = D_MODEL // 32


def _route_body(x_ref, wg_ref, xbf_ref, e1_ref, e2_ref, w1_ref, w2_ref,
                r1_ref, r2_ref, cnt_ref, carry_ref):
    i = pl.program_id(0)
    xt = x_ref[...]
    T = xt.shape[0]
    H = D_MODEL // 2
    xbf_ref[...] = pltpu.pack_elementwise(
        [xt[:, :H], xt[:, H:]], packed_dtype=jnp.bfloat16)

    logits = lax.dot_general(wg_ref[...], xt, (((1,), (1,)), ((), ())),
                             preferred_element_type=jnp.float32)

    iota = lax.broadcasted_iota(jnp.int32, (E, T), 0)
    m1 = jnp.max(logits, axis=0, keepdims=True)
    a1 = jnp.min(jnp.where(logits >= m1, iota, E), axis=0, keepdims=True)
    masked = jnp.where(iota == a1, -jnp.inf, logits)
    m2 = jnp.max(masked, axis=0, keepdims=True)
    a2 = jnp.min(jnp.where(masked >= m2, iota, E), axis=0, keepdims=True)

    w1 = 1.0 / (1.0 + jnp.exp(m2 - m1))
    w2 = 1.0 - w1

    iota16 = lax.broadcasted_iota(jnp.int32, (2 * E, T), 0)
    oh1 = (iota16 == a1).astype(jnp.float32)
    oh2 = (iota16 == a2).astype(jnp.float32)
    oh = oh1 + oh2

    tri = (lax.broadcasted_iota(jnp.int32, (T, T), 0) <
           lax.broadcasted_iota(jnp.int32, (T, T), 1)).astype(jnp.float32)
    csum = lax.dot_general(oh, tri, (((1,), (0,)), ((), ())),
                           preferred_element_type=jnp.float32)

    @pl.when(i == 0)
    def _():
        carry_ref[...] = jnp.zeros_like(carry_ref)

    carry = carry_ref[...]
    r1 = jnp.sum(oh1 * (carry + csum), axis=0, keepdims=True)
    r2 = jnp.sum(oh2 * (carry + csum), axis=0, keepdims=True)
    carry_ref[...] = carry + jnp.sum(oh, axis=1, keepdims=True)

    e1_ref[...] = a1.reshape(1, 1, T)
    e2_ref[...] = a2.reshape(1, 1, T)
    w1_ref[...] = w1.reshape(1, 1, T)
    w2_ref[...] = w2.reshape(1, 1, T)
    r1_ref[...] = r1.astype(jnp.int32).reshape(1, 1, T)
    r2_ref[...] = r2.astype(jnp.int32).reshape(1, 1, T)

    ones = jnp.ones((1, T), jnp.float32)
    ctile = lax.dot_general(ones, oh, (((1,), (1,)), ((), ())),
                            preferred_element_type=jnp.float32)

    @pl.when(i == 0)
    def _():
        cnt_ref[...] = jnp.zeros_like(cnt_ref)

    cnt_ref[...] = cnt_ref[...] + ctile.astype(jnp.int32).reshape(1, 1, 2 * E)


def _route(x2d, Wg):
    T = TOK_TILE
    specs3 = pl.BlockSpec((1, 1, T), lambda i: (i, 0, 0))
    shape3i = jax.ShapeDtypeStruct((N_TOK_TILES, 1, T), jnp.int32)
    shape3f = jax.ShapeDtypeStruct((N_TOK_TILES, 1, T), jnp.float32)
    return pl.pallas_call(
        _route_body,
        grid=(N_TOK_TILES,),
        in_specs=[
            pl.BlockSpec((T, D_MODEL), lambda i: (i, 0)),
            pl.BlockSpec((E, D_MODEL), lambda i: (0, 0)),
        ],
        out_specs=[pl.BlockSpec((T, D_MODEL // 2), lambda i: (i, 0)),
                   specs3, specs3, specs3, specs3, specs3, specs3,
                   pl.BlockSpec((1, 1, 2 * E), lambda i: (0, 0, 0))],
        out_shape=[jax.ShapeDtypeStruct((SEQ, D_MODEL // 2), jnp.int32),
                   shape3i, shape3i, shape3f, shape3f, shape3i, shape3i,
                   jax.ShapeDtypeStruct((1, 1, 2 * E), jnp.int32)],
        scratch_shapes=[pltpu.VMEM((2 * E, 1), jnp.float32)],
        compiler_params=pltpu.CompilerParams(
            dimension_semantics=("arbitrary",)),
    )(x2d, Wg)


def _dispatch_body(x_hbm, e1_hbm, e2_hbm, r1_hbm, r2_hbm, cnt_hbm,
                   xs_hbm, pos1_hbm, pos2_hbm, eid_hbm,
                   ev1, ev2, rv1, rv2, cnt_v, offp_v, p1_v, p2_v,
                   delta_v, eid_v, xrows, sem, xsem):
    wid = lax.axis_index("s") * 2 + lax.axis_index("c")
    base = wid * TPW

    xdma = pltpu.async_copy(x_hbm.at[pl.ds(base, TPW)], xrows, xsem)

    c1 = pltpu.async_copy(e1_hbm.at[pl.ds(base, TPW)], ev1, sem)
    c2 = pltpu.async_copy(e2_hbm.at[pl.ds(base, TPW)], ev2, sem)
    c3 = pltpu.async_copy(r1_hbm.at[pl.ds(base, TPW)], rv1, sem)
    c4 = pltpu.async_copy(r2_hbm.at[pl.ds(base, TPW)], rv2, sem)
    c5 = pltpu.async_copy(cnt_hbm, cnt_v, sem)
    c1.wait(); c2.wait(); c3.wait(); c4.wait(); c5.wait()

    c = cnt_v[...]
    pc = ((c + (ROW_TILE - 1)) >> 8) << 8
    incl = jnp.cumsum(pc)
    offp = incl - pc
    offp_v[...] = offp

    for v in range(NVEC):
        sl = pl.ds(16 * v, 16)
        p1_v[sl] = plsc.load_gather(offp_v, [ev1[sl]]) + rv1[sl]
        p2_v[sl] = plsc.load_gather(offp_v, [ev2[sl]]) + rv2[sl]

    pltpu.sync_copy(p1_v, pos1_hbm.at[pl.ds(base, TPW)])
    pltpu.sync_copy(p2_v, pos2_hbm.at[pl.ds(base, TPW)])

    @pl.when(wid == 0)
    def _():
        zeros = jnp.zeros((16,), jnp.int32)
        delta_v[pl.ds(0, 16)] = zeros
        delta_v[pl.ds(16, 16)] = zeros
        ts = offp >> 8
        lane = lax.broadcasted_iota(jnp.int32, (16,), 0)
        mask = (lane >= 1) & (lane < E)
        plsc.addupdate_scatter(delta_v, [ts], jnp.ones((16,), jnp.int32),
                               mask=mask)
        d0 = delta_v[pl.ds(0, 16)]
        c0 = jnp.cumsum(d0)
        s0 = jnp.sum(d0, axis=0)
        c1 = jnp.cumsum(delta_v[pl.ds(16, 16)]) + s0
        eid_v[pl.ds(0, 16)] = c0
        eid_v[pl.ds(16, 16)] = c1
        pltpu.sync_copy(eid_v, eid_hbm)

    xdma.wait()
    s1 = pltpu.async_copy(xrows, xs_hbm.at[p1_v], sem)
    s2 = pltpu.async_copy(xrows, xs_hbm.at[p2_v], sem)
    s1.wait()
    s2.wait()


def _dispatch(xbf, e1, e2, r1, r2, cnt):
    mesh = plsc.VectorSubcoreMesh(core_axis_name="c", subcore_axis_name="s",
                                  num_cores=2, num_subcores=16)
    f = pl.kernel(
        _dispatch_body,
        out_type=[
            jax.ShapeDtypeStruct((CAP, D_MODEL // 2), jnp.int32),
            jax.ShapeDtypeStruct((SEQ,), jnp.int32),
            jax.ShapeDtypeStruct((SEQ,), jnp.int32),
            jax.ShapeDtypeStruct((32,), jnp.int32),
        ],
        mesh=mesh,
        scratch_types=[
            pltpu.VMEM((TPW,), jnp.int32),
            pltpu.VMEM((TPW,), jnp.int32),
            pltpu.VMEM((TPW,), jnp.int32),
            pltpu.VMEM((TPW,), jnp.int32),
            pltpu.VMEM((16,), jnp.int32),
            pltpu.VMEM((16,), jnp.int32),
            pltpu.VMEM((TPW,), jnp.int32),
            pltpu.VMEM((TPW,), jnp.int32),
            pltpu.VMEM((32,), jnp.int32),
            pltpu.VMEM((32,), jnp.int32),
            pltpu.VMEM((TPW, D_MODEL // 2), jnp.int32),
            pltpu.SemaphoreType.DMA,
            pltpu.SemaphoreType.DMA,
        ],
        compiler_params=pltpu.CompilerParams(needs_layout_passes=False),
    )
    return f(xbf, e1, e2, r1, r2, cnt)


def _expert_mm_body(eid_ref, xs_ref, we_ref, be_ref, ys_ref):
    i = pl.program_id(0)
    H = D_MODEL // 2
    w = xs_ref[...]
    xlo = pltpu.unpack_elementwise(
        w, index=0, packed_dtype=jnp.bfloat16, unpacked_dtype=jnp.float32)
    xhi = pltpu.unpack_elementwise(
        w, index=1, packed_dtype=jnp.bfloat16, unpacked_dtype=jnp.float32)
    xf = jnp.concatenate([xlo, xhi], axis=1)
    for j in range(MM_TILES):
        e = eid_ref[MM_TILES * i + j]
        sl = pl.ds(ROW_TILE * j, ROW_TILE)
        y = lax.dot_general(
            xf[ROW_TILE * j:ROW_TILE * (j + 1)], we_ref[e],
            (((1,), (1,)), ((), ())),
            preferred_element_type=jnp.float32) + be_ref[pl.ds(e, 1), :]
        ys_ref[sl] = pltpu.pack_elementwise(
            [y[:, :H], y[:, H:]], packed_dtype=jnp.bfloat16)


def _expert_mm(eid, xs, We, be):
    grid_spec = pltpu.PrefetchScalarGridSpec(
        num_scalar_prefetch=1,
        grid=(N_ROW_TILES // MM_TILES,),
        in_specs=[
            pl.BlockSpec((MM_TILES * ROW_TILE, D_MODEL // 2),
                         lambda i, eid: (i, 0)),
            pl.BlockSpec((E, D_MODEL, D_MODEL), lambda i, eid: (0, 0, 0)),
            pl.BlockSpec((E, D_MODEL), lambda i, eid: (0, 0)),
        ],
        out_specs=pl.BlockSpec((MM_TILES * ROW_TILE, D_MODEL // 2),
                               lambda i, eid: (i, 0)),
    )
    return pl.pallas_call(
        _expert_mm_body,
        grid_spec=grid_spec,
        out_shape=jax.ShapeDtypeStruct((CAP, D_MODEL // 2), jnp.int32),
        compiler_params=pltpu.CompilerParams(
            dimension_semantics=("arbitrary",)),
    )(eid, xs, We, be)


def _combine_body(ys_hbm, pos1_hbm, pos2_hbm, w1_hbm, w2_hbm, out_hbm,
                  p1_v, p2_v, w1_v, w2_v, buf1, buf2, obuf, sem, sem2):
    wid = lax.axis_index("s") * 2 + lax.axis_index("c")
    base = wid * TPW

    c1 = pltpu.async_copy(pos1_hbm.at[pl.ds(base, TPW)], p1_v, sem)
    c2 = pltpu.async_copy(pos2_hbm.at[pl.ds(base, TPW)], p2_v, sem)
    c3 = pltpu.async_copy(w1_hbm.at[pl.ds(base, TPW)], w1_v, sem)
    c4 = pltpu.async_copy(w2_hbm.at[pl.ds(base, TPW)], w2_v, sem)
    c1.wait(); c2.wait(); c3.wait(); c4.wait()

    g1 = pltpu.async_copy(ys_hbm.at[p1_v], buf1, sem)
    g2 = pltpu.async_copy(ys_hbm.at[p2_v], buf2, sem2)
    g1.wait()
    g2.wait()

    hmask = jnp.int32(-65536)
    H = D_MODEL // 2

    @plsc.parallel_loop(0, TPW, 1, unroll=4)
    def tok_body(t):
        t16 = jnp.full((16,), t, jnp.int32)
        wt1 = plsc.load_gather(w1_v, [t16])
        wt2 = plsc.load_gather(w2_v, [t16])
        for g in range(DW):
            sl = pl.ds(16 * g, 16)
            v1 = buf1[t, sl]
            v2 = buf2[t, sl]
            lo1 = plsc.bitcast(v1 << 16, jnp.float32)
            hi1 = plsc.bitcast(v1 & hmask, jnp.float32)
            lo2 = plsc.bitcast(v2 << 16, jnp.float32)
            hi2 = plsc.bitcast(v2 & hmask, jnp.float32)
            obuf[t, pl.ds(16 * g, 16)] = wt1 * lo1 + wt2 * lo2
            obuf[t, pl.ds(H + 16 * g, 16)] = wt1 * hi1 + wt2 * hi2
    pltpu.sync_copy(obuf, out_hbm.at[pl.ds(base, TPW)])


def _combine(ys_i32, pos1, pos2, w1, w2):
    mesh = plsc.VectorSubcoreMesh(core_axis_name="c", subcore_axis_name="s",
                                  num_cores=2, num_subcores=16)
    f = pl.kernel(
        _combine_body,
        out_type=jax.ShapeDtypeStruct((SEQ, D_MODEL), jnp.float32),
        mesh=mesh,
        scratch_types=[
            pltpu.VMEM((TPW,), jnp.int32),
            pltpu.VMEM((TPW,), jnp.int32),
            pltpu.VMEM((TPW,), jnp.float32),
            pltpu.VMEM((TPW,), jnp.float32),
            pltpu.VMEM((TPW, D_MODEL // 2), jnp.int32),
            pltpu.VMEM((TPW, D_MODEL // 2), jnp.int32),
            pltpu.VMEM((TPW, D_MODEL), jnp.float32),
            pltpu.SemaphoreType.DMA,
            pltpu.SemaphoreType.DMA,
        ],
        compiler_params=pltpu.CompilerParams(needs_layout_passes=False),
    )
    return f(ys_i32, pos1, pos2, w1, w2)


@jax.jit
def _moe(x2d, Wg, We, be):
    xpk, e1, e2, w1, w2, r1, r2, cnt = _route(x2d, Wg)
    flat = lambda a: a.reshape(-1)
    xs_pk, pos1, pos2, eid = _dispatch(
        xpk, flat(e1), flat(e2), flat(r1), flat(r2), flat(cnt))
    ys_pk = _expert_mm(eid[:N_ROW_TILES], xs_pk, We, be)
    return _combine(ys_pk, pos1, pos2, flat(w1), flat(w2))


def kernel(x, Wg, We, be):
    B, S, D = x.shape
    out = _moe(x.reshape(S, D), Wg, We, be)
    return out.reshape(B, S, D)
